# baseline (speedup 1.0000x reference)
import functools

import jax
import jax.numpy as jnp
from jax import lax
from jax.experimental import pallas as pl
from jax.experimental.pallas import tpu as pltpu
from jax.experimental.pallas import tpu_sc as plsc

N = 10000
E = 160000
IN = 1024
H = 256
H2 = 128
OUT = 4
OUTP = 16

NC = 2
NS = 16
NW = NC * NS
RPW = 320
NPAD = NW * RPW

B = 128
VW = 2048
EPD = 163840
NWIN = EPD // VW
CAP_DATA = EPD + 128
CAP = CAP_DATA + 128

MBLK = 1000
MB = 5


def _tc_layer(a, W_rel, W_root, b, first):
    Fin = W_rel.shape[0]
    F = W_rel.shape[1]

    def body(a_ref, wr_ref, wo_ref, b_ref, y_ref, z_ref):
        h = a_ref[...]
        if not first:
            h = jnp.maximum(h, 0.0)
        y_ref[...] = jnp.dot(h, wr_ref[...], preferred_element_type=jnp.float32)
        z_ref[...] = (
            jnp.dot(h, wo_ref[...], preferred_element_type=jnp.float32) + b_ref[...]
        )

    return pl.pallas_call(
        body,
        grid=(NC * MB,),
        in_specs=[
            pl.BlockSpec((MBLK, Fin), lambda m: (m, 0)),
            pl.BlockSpec((Fin, F), lambda m: (0, 0)),
            pl.BlockSpec((Fin, F), lambda m: (0, 0)),
            pl.BlockSpec((1, F), lambda m: (0, 0)),
        ],
        out_specs=[
            pl.BlockSpec((MBLK, F), lambda m: (m, 0)),
            pl.BlockSpec((MBLK, F), lambda m: (m, 0)),
        ],
        out_shape=[
            jax.ShapeDtypeStruct((N, F), jnp.float32),
            jax.ShapeDtypeStruct((NPAD, F), jnp.float32),
        ],
    )(a, W_rel, W_root, b.reshape(1, F))


def _make_partition():
    mesh = plsc.VectorSubcoreMesh(core_axis_name="c", subcore_axis_name="s")

    @functools.partial(
        pl.kernel,
        out_type=[
            jax.ShapeDtypeStruct((NW * CAP,), jnp.int32),
            jax.ShapeDtypeStruct((NW * CAP,), jnp.int32),
            jax.ShapeDtypeStruct((NW * CAP,), jnp.float32),
            jax.ShapeDtypeStruct((NW * 16,), jnp.int32),
        ],
        mesh=mesh,
        scratch_types=[
            pltpu.VMEM((VW,), jnp.int32),
            pltpu.VMEM((VW,), jnp.int32),
            pltpu.VMEM((VW,), jnp.float32),
            pltpu.VMEM((VW,), jnp.int32),
            pltpu.VMEM((VW,), jnp.int32),
            pltpu.VMEM((VW,), jnp.float32),
            pltpu.VMEM((VW,), jnp.int32),
            pltpu.VMEM((VW // 128, 128), jnp.int32),
            pltpu.VMEM((24,), jnp.int32),
            pltpu.VMEM((16,), jnp.int32),
            pltpu.SemaphoreType.DMA,
            pltpu.SemaphoreType.DMA,
        ],
    )
    def partition(src_hbm, dst_hbm, w_hbm, osrc, otgt, ow, ocnt,
                  sA, dA, wA, sB, dB, wB, tbuf, pos2d, pbuf, cbuf,
                  sem, lsem):
        c = lax.axis_index("c")
        s = lax.axis_index("s")
        w = c * NS + s
        lo = w * RPW
        base = w * CAP
        lane = lax.iota(jnp.int32, 16)

        pbuf[pl.ds(0, 16)] = jnp.zeros((16,), jnp.int32)

        def load(win, bufs):
            wbase = pl.multiple_of(win * VW, 8)
            return [
                pltpu.async_copy(src_hbm.at[pl.ds(wbase, VW)], bufs[0], lsem),
                pltpu.async_copy(dst_hbm.at[pl.ds(wbase, VW)], bufs[1], lsem),
                pltpu.async_copy(w_hbm.at[pl.ds(wbase, VW)], bufs[2], lsem),
            ]

        def process(bufs, off):
            ws_, wd_, ww_ = bufs
            descs = []
            for jb in range(VW // 128):
                def grp(gg, off, jb=jb):
                    i = jb * 128 + gg * 16
                    d = wd_[pl.ds(i, 16)]
                    m = (d >= lo) & (d < lo + RPW)
                    v = jnp.where(m, jnp.int32(1), jnp.int32(0))
                    for kk in (1, 2, 4, 8):
                        pbuf[pl.ds(8, 16)] = v
                        v = v + pbuf[pl.ds(8 - kk, 16)]
                    tbuf[pl.ds(i, 16)] = d - lo
                    slot = jb * 0 + gg * 16 + lane
                    pos2d[jb, pl.ds(gg * 16, 16)] = jnp.where(
                        m, base + off + v - 1, base + CAP_DATA + slot
                    )
                    return off + v[15]
                off = lax.fori_loop(0, 8, grp, off)
                descs.append(pltpu.async_copy(
                    ws_.at[pl.ds(jb * 128, 128)], osrc.at[pos2d.at[jb]], sem))
                descs.append(pltpu.async_copy(
                    tbuf.at[pl.ds(jb * 128, 128)], otgt.at[pos2d.at[jb]], sem))
                descs.append(pltpu.async_copy(
                    ww_.at[pl.ds(jb * 128, 128)], ow.at[pos2d.at[jb]], sem))
            for dsc in descs:
                dsc.wait()
            return off

        bufsA = (sA, dA, wA)
        bufsB = (sB, dB, wB)
        for dsc in load(0, bufsA):
            dsc.wait()

        def pair(pp, off):
            descB = load(2 * pp + 1, bufsB)
            off = process(bufsA, off)
            for dsc in descB:
                dsc.wait()
            descA = load(jnp.minimum(2 * pp + 2, NWIN - 1), bufsA)
            off = process(bufsB, off)
            for dsc in descA:
                dsc.wait()
            return off

        off = lax.fori_loop(0, NWIN // 2, pair, 0)

        padc = (128 - lax.rem(off, 128)) % 128
        for gg in range(8):
            sA[pl.ds(gg * 16, 16)] = lo + lane
            dA[pl.ds(gg * 16, 16)] = jnp.zeros((16,), jnp.int32)
            wA[pl.ds(gg * 16, 16)] = jnp.zeros((16,), jnp.float32)
            slot = gg * 16 + lane
            pos2d[0, pl.ds(gg * 16, 16)] = jnp.where(
                slot < padc, base + off + slot, base + CAP_DATA + slot
            )
        pltpu.sync_copy(sA.at[pl.ds(0, 128)], osrc.at[pos2d.at[0]])
        pltpu.sync_copy(dA.at[pl.ds(0, 128)], otgt.at[pos2d.at[0]])
        pltpu.sync_copy(wA.at[pl.ds(0, 128)], ow.at[pos2d.at[0]])

        nbat = (off + 127) // 128
        cbuf[pl.ds(0, 16)] = jnp.broadcast_to(nbat, (16,))
        pltpu.sync_copy(cbuf, ocnt.at[pl.ds(pl.multiple_of(w * 16, 8), 16)])

    return partition


@functools.cache
def _make_sc_layer(F, relu_gather=False, zero_init=False):
    mesh = plsc.VectorSubcoreMesh(core_axis_name="c", subcore_axis_name="s")

    @functools.partial(
        pl.kernel,
        out_type=jax.ShapeDtypeStruct((NPAD, F), jnp.float32),
        mesh=mesh,
        scratch_types=[
            pltpu.VMEM((RPW, F), jnp.float32),
            pltpu.VMEM((B,), jnp.int32),
            pltpu.VMEM((B,), jnp.int32),
            pltpu.VMEM((B,), jnp.float32),
            pltpu.VMEM((B, F), jnp.float32),
            pltpu.VMEM((16,), jnp.int32),
            pltpu.SemaphoreType.DMA,
        ],
    )
    def sc_layer(y_hbm, z_hbm, lsrc, ltgt, lw, lcnt, out_hbm,
                 acc, src_b, tgt_b, w_b, rows, cbuf, sem):
        c = lax.axis_index("c")
        s = lax.axis_index("s")
        w = c * NS + s
        base = w * CAP

        if zero_init:
            zv = jnp.zeros((16,), jnp.float32)

            def zrow(r, _):
                for q in range(F // 16):
                    acc[r, pl.ds(q * 16, 16)] = zv
                return 0
            lax.fori_loop(0, RPW, zrow, 0)
        else:
            pltpu.sync_copy(z_hbm.at[pl.ds(pl.multiple_of(w * RPW, 8), RPW)], acc)
        pltpu.sync_copy(lcnt.at[pl.ds(pl.multiple_of(w * 16, 8), 16)], cbuf)
        nbat = cbuf[pl.ds(0, 16)][0]

        def batch(k, _):
            pltpu.sync_copy(lsrc.at[pl.ds(pl.multiple_of(base + k * B, 8), B)], src_b)
            pltpu.sync_copy(ltgt.at[pl.ds(pl.multiple_of(base + k * B, 8), B)], tgt_b)
            pltpu.sync_copy(lw.at[pl.ds(pl.multiple_of(base + k * B, 8), B)], w_b)
            pltpu.async_copy(y_hbm.at[src_b], rows, sem).wait()

            def grp(g, _):
                tv = tgt_b[pl.ds(g * 16, 16)]
                wv = w_b[pl.ds(g * 16, 16)]
                for t16 in range(16):
                    t = tv[t16]
                    ws = wv[t16]
                    e = g * 16 + t16
                    for q in range(F // 16):
                        r = rows[e, pl.ds(q * 16, 16)]
                        if relu_gather:
                            r = jnp.maximum(r, 0.0)
                        acc[t, pl.ds(q * 16, 16)] = (
                            acc[t, pl.ds(q * 16, 16)] + r * ws
                        )
                return 0
            lax.fori_loop(0, B // 16, grp, 0)
            return 0
        lax.fori_loop(0, nbat, batch, 0)

        pltpu.sync_copy(acc, out_hbm.at[pl.ds(pl.multiple_of(w * RPW, 8), RPW)])

    return sc_layer


def _tc_final(g3, a2, W_rel, W_root, b):
    def body(g_ref, a_ref, wr_ref, wo_ref, b_ref, o_ref):
        h = jnp.maximum(a_ref[...], 0.0)
        o_ref[...] = (
            jnp.dot(g_ref[...], wr_ref[...], preferred_element_type=jnp.float32)
            + jnp.dot(h, wo_ref[...], preferred_element_type=jnp.float32)
            + b_ref[...]
        )

    return pl.pallas_call(
        body,
        grid=(NC * MB,),
        in_specs=[
            pl.BlockSpec((MBLK, H2), lambda m: (m, 0)),
            pl.BlockSpec((MBLK, H2), lambda m: (m, 0)),
            pl.BlockSpec((H2, OUT), lambda m: (0, 0)),
            pl.BlockSpec((H2, OUT), lambda m: (0, 0)),
            pl.BlockSpec((1, OUT), lambda m: (0, 0)),
        ],
        out_specs=pl.BlockSpec((MBLK, OUT), lambda m: (m, 0)),
        out_shape=jax.ShapeDtypeStruct((N, OUT), jnp.float32),
    )(g3, a2, W_rel, W_root, b.reshape(1, OUT))


def kernel(x, edge_index, edge_weight, batch,
           W1_rel, b1, W1_root, W2_rel, b2, W2_root, W3_rel, b3, W3_root):
    src = edge_index[0].astype(jnp.int32)
    dst = edge_index[1].astype(jnp.int32)
    w = edge_weight.astype(jnp.float32)
    pad = EPD - E
    src = jnp.pad(src, (0, pad))
    dst = jnp.pad(dst, (0, pad), constant_values=jnp.int32(1 << 29))
    w = jnp.pad(w, (0, pad))

    lsrc, ltgt, lw, lcnt = _make_partition()(src, dst, w)

    y1, z1 = _tc_layer(x, W1_rel, W1_root, b1, first=True)
    a1 = _make_sc_layer(H)(y1, z1, lsrc, ltgt, lw, lcnt)
    y2, z2 = _tc_layer(a1, W2_rel, W2_root, b2, first=False)
    a2 = _make_sc_layer(H2)(y2, z2, lsrc, ltgt, lw, lcnt)
    g3 = _make_sc_layer(H2, relu_gather=True, zero_init=True)(
        a2, a2, lsrc, ltgt, lw, lcnt)
    return _tc_final(g3[:N], a2[:N], W3_rel, W3_root, b3)


# speedup vs baseline: 56.1014x; 56.1014x over previous
import functools

import jax
import jax.numpy as jnp
from jax import lax
from jax.experimental import pallas as pl
from jax.experimental.pallas import tpu as pltpu
from jax.experimental.pallas import tpu_sc as plsc

N = 10000
E = 160000
IN = 1024
H = 256
H2 = 128
OUT = 4
OUTP = 16

NC = 2
NS = 16
NW = NC * NS
RPW = 320
NPAD = NW * RPW

B = 128
NBAT = E // B

MBLK = 1000
MB = 5


def _tc_layer(a, W_rel, W_root, b, first):
    Fin = W_rel.shape[0]
    F = W_rel.shape[1]

    def body(a_ref, wr_ref, wo_ref, b_ref, y_ref, z_ref):
        h = a_ref[...]
        if not first:
            h = jnp.maximum(h, 0.0)
        y_ref[...] = jnp.dot(h, wr_ref[...], preferred_element_type=jnp.float32)
        z_ref[...] = (
            jnp.dot(h, wo_ref[...], preferred_element_type=jnp.float32) + b_ref[...]
        )

    return pl.pallas_call(
        body,
        grid=(NC * MB,),
        in_specs=[
            pl.BlockSpec((MBLK, Fin), lambda m: (m, 0)),
            pl.BlockSpec((Fin, F), lambda m: (0, 0)),
            pl.BlockSpec((Fin, F), lambda m: (0, 0)),
            pl.BlockSpec((1, F), lambda m: (0, 0)),
        ],
        out_specs=[
            pl.BlockSpec((MBLK, F), lambda m: (m, 0)),
            pl.BlockSpec((MBLK, F), lambda m: (m, 0)),
        ],
        out_shape=[
            jax.ShapeDtypeStruct((N, F), jnp.float32),
            jax.ShapeDtypeStruct((NPAD, F), jnp.float32),
        ],
    )(a, W_rel, W_root, b.reshape(1, F))


@functools.cache
def _make_sc_layer(F, relu_gather=False, zero_init=False):
    mesh = plsc.VectorSubcoreMesh(core_axis_name="c", subcore_axis_name="s")

    @functools.partial(
        pl.kernel,
        out_type=jax.ShapeDtypeStruct((NPAD, F), jnp.float32),
        mesh=mesh,
        scratch_types=[
            pltpu.VMEM((RPW, F), jnp.float32),
            pltpu.VMEM((B,), jnp.int32),
            pltpu.VMEM((B,), jnp.int32),
            pltpu.VMEM((B,), jnp.float32),
            pltpu.VMEM((B, F), jnp.float32),
            pltpu.VMEM((16,), jnp.int32),
            pltpu.SemaphoreType.DMA,
        ],
    )
    def sc_layer(y_hbm, z_hbm, lsrc, ldst, lw, bnd, out_hbm,
                 acc, src_b, dst_b, w_b, rows, bbuf, sem):
        c = lax.axis_index("c")
        s = lax.axis_index("s")
        w = c * NS + s
        lo = w * RPW

        if zero_init:
            zv = jnp.zeros((16,), jnp.float32)

            def zrow(r, _):
                for q in range(F // 16):
                    acc[r, pl.ds(q * 16, 16)] = zv
                return 0
            lax.fori_loop(0, RPW, zrow, 0)
        else:
            pltpu.sync_copy(z_hbm.at[pl.ds(pl.multiple_of(w * RPW, 8), RPW)], acc)

        pltpu.sync_copy(bnd.at[pl.ds(pl.multiple_of(w * 16, 8), 16)], bbuf)
        st = bbuf[pl.ds(0, 16)][0]
        pltpu.sync_copy(bnd.at[pl.ds(pl.multiple_of((w + 1) * 16, 8), 16)], bbuf)
        en = bbuf[pl.ds(0, 16)][0]
        kb0 = st // B
        kb1 = (en + B - 1) // B

        def batch(k, _):
            kw = pl.multiple_of(k * B, 8)
            pltpu.sync_copy(lsrc.at[pl.ds(kw, B)], src_b)
            pltpu.sync_copy(ldst.at[pl.ds(kw, B)], dst_b)
            pltpu.sync_copy(lw.at[pl.ds(kw, B)], w_b)
            pltpu.async_copy(y_hbm.at[src_b], rows, sem).wait()

            def grp(g, _):
                d = dst_b[pl.ds(g * 16, 16)]
                m = (d >= lo) & (d < lo + RPW)
                tv = jnp.where(m, d - lo, 0)
                wv = jnp.where(m, w_b[pl.ds(g * 16, 16)], 0.0)
                for t16 in range(16):
                    t = tv[t16]
                    ws = wv[t16]
                    e = g * 16 + t16
                    for q in range(F // 16):
                        r = rows[e, pl.ds(q * 16, 16)]
                        if relu_gather:
                            r = jnp.maximum(r, 0.0)
                        acc[t, pl.ds(q * 16, 16)] = (
                            acc[t, pl.ds(q * 16, 16)] + r * ws
                        )
                return 0
            lax.fori_loop(0, B // 16, grp, 0)
            return 0
        lax.fori_loop(kb0, kb1, batch, 0)

        pltpu.sync_copy(acc, out_hbm.at[pl.ds(pl.multiple_of(w * RPW, 8), RPW)])

    return sc_layer


def _tc_final(g3, a2, W_rel, W_root, b):
    def body(g_ref, a_ref, wr_ref, wo_ref, b_ref, o_ref):
        h = jnp.maximum(a_ref[...], 0.0)
        o_ref[...] = (
            jnp.dot(g_ref[...], wr_ref[...], preferred_element_type=jnp.float32)
            + jnp.dot(h, wo_ref[...], preferred_element_type=jnp.float32)
            + b_ref[...]
        )

    return pl.pallas_call(
        body,
        grid=(NC * MB,),
        in_specs=[
            pl.BlockSpec((MBLK, H2), lambda m: (m, 0)),
            pl.BlockSpec((MBLK, H2), lambda m: (m, 0)),
            pl.BlockSpec((H2, OUT), lambda m: (0, 0)),
            pl.BlockSpec((H2, OUT), lambda m: (0, 0)),
            pl.BlockSpec((1, OUT), lambda m: (0, 0)),
        ],
        out_specs=pl.BlockSpec((MBLK, OUT), lambda m: (m, 0)),
        out_shape=jax.ShapeDtypeStruct((N, OUT), jnp.float32),
    )(g3, a2, W_rel, W_root, b.reshape(1, OUT))


def kernel(x, edge_index, edge_weight, batch,
           W1_rel, b1, W1_root, W2_rel, b2, W2_root, W3_rel, b3, W3_root):
    src = edge_index[0].astype(jnp.int32)
    dst = edge_index[1].astype(jnp.int32)
    w = edge_weight.astype(jnp.float32)
    order = jnp.argsort(dst)
    lsrc = src[order]
    ldst = dst[order]
    lw = w[order]
    bounds = jnp.searchsorted(
        ldst, (jnp.arange(NW + 1) * RPW).astype(jnp.int32)).astype(jnp.int32)
    bnd = jnp.repeat(bounds, 16)

    y1, z1 = _tc_layer(x, W1_rel, W1_root, b1, first=True)
    a1 = _make_sc_layer(H)(y1, z1, lsrc, ldst, lw, bnd)
    y2, z2 = _tc_layer(a1, W2_rel, W2_root, b2, first=False)
    a2 = _make_sc_layer(H2)(y2, z2, lsrc, ldst, lw, bnd)
    g3 = _make_sc_layer(H2, relu_gather=True, zero_init=True)(
        a2, a2, lsrc, ldst, lw, bnd)
    return _tc_final(g3[:N], a2[:N], W3_rel, W3_root, b3)


# speedup vs baseline: 62.3119x; 1.1107x over previous
import functools

import jax
import jax.numpy as jnp
from jax import lax
from jax.experimental import pallas as pl
from jax.experimental.pallas import tpu as pltpu
from jax.experimental.pallas import tpu_sc as plsc

N = 10000
E = 160000
IN = 1024
H = 256
H2 = 128
OUT = 4
OUTP = 16

NC = 2
NS = 16
NW = NC * NS
RPW = 320
NPAD = NW * RPW

B = 128
NBAT = E // B

MBLK = 1000
MB = 5


def _tc_layer(a, W_rel, W_root, b, first):
    Fin = W_rel.shape[0]
    F = W_rel.shape[1]

    def body(a_ref, wr_ref, wo_ref, b_ref, y_ref, z_ref):
        h = a_ref[...]
        if not first:
            h = jnp.maximum(h, 0.0)
        y_ref[...] = jnp.dot(h, wr_ref[...], preferred_element_type=jnp.float32)
        z_ref[...] = (
            jnp.dot(h, wo_ref[...], preferred_element_type=jnp.float32) + b_ref[...]
        )

    return pl.pallas_call(
        body,
        grid=(NC * MB,),
        in_specs=[
            pl.BlockSpec((MBLK, Fin), lambda m: (m, 0)),
            pl.BlockSpec((Fin, F), lambda m: (0, 0)),
            pl.BlockSpec((Fin, F), lambda m: (0, 0)),
            pl.BlockSpec((1, F), lambda m: (0, 0)),
        ],
        out_specs=[
            pl.BlockSpec((MBLK, F), lambda m: (m, 0)),
            pl.BlockSpec((MBLK, F), lambda m: (m, 0)),
        ],
        out_shape=[
            jax.ShapeDtypeStruct((N, F), jnp.float32),
            jax.ShapeDtypeStruct((NPAD, F), jnp.float32),
        ],
    )(a, W_rel, W_root, b.reshape(1, F))


@functools.cache
def _make_sc_layer(F, relu_gather=False, zero_init=False):
    BL = 64 if F >= 256 else 128
    CHB = 32
    CH = CHB * BL
    mesh = plsc.VectorSubcoreMesh(core_axis_name="c", subcore_axis_name="s")

    @functools.partial(
        pl.kernel,
        out_type=jax.ShapeDtypeStruct((NPAD, F), jnp.float32),
        mesh=mesh,
        scratch_types=[
            pltpu.VMEM((RPW, F), jnp.float32),
            pltpu.VMEM((CH,), jnp.int32),
            pltpu.VMEM((CH,), jnp.int32),
            pltpu.VMEM((CH,), jnp.float32),
            pltpu.VMEM((BL, F), jnp.float32),
            pltpu.VMEM((BL, F), jnp.float32),
            pltpu.VMEM((16,), jnp.int32),
            pltpu.SemaphoreType.DMA,
            pltpu.SemaphoreType.DMA,
        ],
    )
    def sc_layer(y_hbm, z_hbm, lsrc, ldst, lw, bnd, out_hbm,
                 acc, src_c, dst_c, w_c, rows0, rows1, bbuf, sem0, sem1):
        c = lax.axis_index("c")
        s = lax.axis_index("s")
        w = c * NS + s
        lo = w * RPW

        if zero_init:
            zv = jnp.zeros((16,), jnp.float32)

            def zrow(r, _):
                for q in range(F // 16):
                    acc[r, pl.ds(q * 16, 16)] = zv
                return 0
            lax.fori_loop(0, RPW, zrow, 0)
        else:
            pltpu.sync_copy(z_hbm.at[pl.ds(pl.multiple_of(w * RPW, 8), RPW)], acc)

        pltpu.sync_copy(bnd.at[pl.ds(pl.multiple_of(w * 16, 8), 16)], bbuf)
        st = bbuf[pl.ds(0, 16)][0]
        pltpu.sync_copy(bnd.at[pl.ds(pl.multiple_of((w + 1) * 16, 8), 16)], bbuf)
        en = bbuf[pl.ds(0, 16)][0]
        kb0 = st // BL
        kb1 = (en + BL - 1) // BL
        nb = kb1 - kb0
        nch = (nb + CHB - 1) // CHB

        def gather(bb, rows, sem):
            return pltpu.async_copy(
                y_hbm.at[src_c.at[pl.ds(bb * BL, BL)]], rows, sem)

        def gwait(rows, sem):
            pltpu.make_async_copy(y_hbm.at[src_c.at[pl.ds(0, BL)]],
                                  rows, sem).wait()

        def accum(rows, bb):
            def grp(g, _):
                d = dst_c[pl.ds(bb * BL + g * 16, 16)]
                m = (d >= lo) & (d < lo + RPW)
                tv = jnp.where(m, d - lo, 0)
                wv = jnp.where(m, w_c[pl.ds(bb * BL + g * 16, 16)], 0.0)
                for t16 in range(16):
                    tt = tv[t16]
                    ws = wv[t16]
                    e = g * 16 + t16
                    for q in range(F // 16):
                        r = rows[e, pl.ds(q * 16, 16)]
                        if relu_gather:
                            r = jnp.maximum(r, 0.0)
                        acc[tt, pl.ds(q * 16, 16)] = (
                            acc[tt, pl.ds(q * 16, 16)] + r * ws
                        )
                return 0
            lax.fori_loop(0, BL // 16, grp, 0)

        def chunk(ci, _):
            cw = pl.multiple_of((kb0 + ci * CHB) * BL, 8)
            pltpu.sync_copy(lsrc.at[pl.ds(cw, CH)], src_c)
            pltpu.sync_copy(ldst.at[pl.ds(cw, CH)], dst_c)
            pltpu.sync_copy(lw.at[pl.ds(cw, CH)], w_c)
            nbc = jnp.minimum(nb - ci * CHB, CHB)

            @pl.when(nbc > 0)
            def _():
                gather(0, rows0, sem0)

            @pl.when(nbc > 1)
            def _():
                gather(1, rows1, sem1)

            def bpair(pb, _):
                b0 = pb * 2

                @pl.when(b0 < nbc)
                def _():
                    gwait(rows0, sem0)
                    accum(rows0, b0)

                    @pl.when(b0 + 2 < nbc)
                    def _():
                        gather(b0 + 2, rows0, sem0)

                @pl.when(b0 + 1 < nbc)
                def _():
                    gwait(rows1, sem1)
                    accum(rows1, b0 + 1)

                    @pl.when(b0 + 3 < nbc)
                    def _():
                        gather(b0 + 3, rows1, sem1)
                return 0
            lax.fori_loop(0, CHB // 2, bpair, 0)
            return 0
        lax.fori_loop(0, nch, chunk, 0)

        pltpu.sync_copy(acc, out_hbm.at[pl.ds(pl.multiple_of(w * RPW, 8), RPW)])

    return sc_layer


def _tc_final(g3, a2, W_rel, W_root, b):
    def body(g_ref, a_ref, wr_ref, wo_ref, b_ref, o_ref):
        h = jnp.maximum(a_ref[...], 0.0)
        o_ref[...] = (
            jnp.dot(g_ref[...], wr_ref[...], preferred_element_type=jnp.float32)
            + jnp.dot(h, wo_ref[...], preferred_element_type=jnp.float32)
            + b_ref[...]
        )

    return pl.pallas_call(
        body,
        grid=(NC * MB,),
        in_specs=[
            pl.BlockSpec((MBLK, H2), lambda m: (m, 0)),
            pl.BlockSpec((MBLK, H2), lambda m: (m, 0)),
            pl.BlockSpec((H2, OUT), lambda m: (0, 0)),
            pl.BlockSpec((H2, OUT), lambda m: (0, 0)),
            pl.BlockSpec((1, OUT), lambda m: (0, 0)),
        ],
        out_specs=pl.BlockSpec((MBLK, OUT), lambda m: (m, 0)),
        out_shape=jax.ShapeDtypeStruct((N, OUT), jnp.float32),
    )(g3, a2, W_rel, W_root, b.reshape(1, OUT))


def kernel(x, edge_index, edge_weight, batch,
           W1_rel, b1, W1_root, W2_rel, b2, W2_root, W3_rel, b3, W3_root):
    src = edge_index[0].astype(jnp.int32)
    dst = edge_index[1].astype(jnp.int32)
    w = edge_weight.astype(jnp.float32)
    order = jnp.argsort(dst)
    lsrc = src[order]
    ldst = dst[order]
    lw = w[order]
    bounds = jnp.searchsorted(
        ldst, (jnp.arange(NW + 1) * RPW).astype(jnp.int32)).astype(jnp.int32)
    bnd = jnp.repeat(bounds, 16)

    y1, z1 = _tc_layer(x, W1_rel, W1_root, b1, first=True)
    a1 = _make_sc_layer(H)(y1, z1, lsrc, ldst, lw, bnd)
    y2, z2 = _tc_layer(a1, W2_rel, W2_root, b2, first=False)
    a2 = _make_sc_layer(H2)(y2, z2, lsrc, ldst, lw, bnd)
    g3 = _make_sc_layer(H2, relu_gather=True, zero_init=True)(
        a2, a2, lsrc, ldst, lw, bnd)
    return _tc_final(g3[:N], a2[:N], W3_rel, W3_root, b3)


# speedup vs baseline: 69.2119x; 1.1107x over previous
import functools

import jax
import jax.numpy as jnp
from jax import lax
from jax.experimental import pallas as pl
from jax.experimental.pallas import tpu as pltpu
from jax.experimental.pallas import tpu_sc as plsc

N = 10000
E = 160000
IN = 1024
H = 256
H2 = 128
OUT = 4
OUTP = 16

NC = 2
NS = 16
NW = NC * NS
RPW = 320
NPAD = NW * RPW

B = 128
NBAT = E // B

MBLK = 1000
MB = 5


def _tc_layer(a, W_rel, W_root, b, first, y_bf16=False):
    Fin = W_rel.shape[0]
    F = W_rel.shape[1]
    ydt = jnp.bfloat16 if y_bf16 else jnp.float32

    def body(a_ref, wr_ref, wo_ref, b_ref, y_ref, z_ref):
        h = a_ref[...]
        if not first:
            h = jnp.maximum(h, 0.0)
        y = jnp.dot(h, wr_ref[...], preferred_element_type=jnp.float32)
        y_ref[...] = y.astype(ydt)
        z_ref[...] = (
            jnp.dot(h, wo_ref[...], preferred_element_type=jnp.float32) + b_ref[...]
        )

    return pl.pallas_call(
        body,
        grid=(NC * MB,),
        in_specs=[
            pl.BlockSpec((MBLK, Fin), lambda m: (m, 0)),
            pl.BlockSpec((Fin, F), lambda m: (0, 0)),
            pl.BlockSpec((Fin, F), lambda m: (0, 0)),
            pl.BlockSpec((1, F), lambda m: (0, 0)),
        ],
        out_specs=[
            pl.BlockSpec((MBLK, F), lambda m: (m, 0)),
            pl.BlockSpec((MBLK, F), lambda m: (m, 0)),
        ],
        out_shape=[
            jax.ShapeDtypeStruct((N, F), ydt),
            jax.ShapeDtypeStruct((NPAD, F), jnp.float32),
        ],
    )(a, W_rel, W_root, b.reshape(1, F))


@functools.cache
def _make_sc_layer(F, relu_gather=False, zero_init=False, bf16_packed=False):
    FW = F // 2 if bf16_packed else F
    BL = 64 if FW >= 256 else 128
    CHB = 32
    CH = CHB * BL
    mesh = plsc.VectorSubcoreMesh(core_axis_name="c", subcore_axis_name="s")

    @functools.partial(
        pl.kernel,
        out_type=jax.ShapeDtypeStruct((NPAD, F), jnp.float32),
        mesh=mesh,
        scratch_types=[
            pltpu.VMEM((RPW, F), jnp.float32),
            pltpu.VMEM((CH,), jnp.int32),
            pltpu.VMEM((CH,), jnp.int32),
            pltpu.VMEM((CH,), jnp.float32),
            pltpu.VMEM((BL, FW), jnp.int32 if bf16_packed else jnp.float32),
            pltpu.VMEM((BL, FW), jnp.int32 if bf16_packed else jnp.float32),
            pltpu.VMEM((16,), jnp.int32),
            pltpu.SemaphoreType.DMA,
            pltpu.SemaphoreType.DMA,
        ],
    )
    def sc_layer(y_hbm, z_hbm, lsrc, ldst, lw, bnd, out_hbm,
                 acc, src_c, dst_c, w_c, rows0, rows1, bbuf, sem0, sem1):
        c = lax.axis_index("c")
        s = lax.axis_index("s")
        w = c * NS + s
        lo = w * RPW

        if zero_init:
            zv = jnp.zeros((16,), jnp.float32)

            def zrow(r, _):
                for q in range(F // 16):
                    acc[r, pl.ds(q * 16, 16)] = zv
                return 0
            lax.fori_loop(0, RPW, zrow, 0)
        else:
            pltpu.sync_copy(z_hbm.at[pl.ds(pl.multiple_of(w * RPW, 8), RPW)], acc)

        pltpu.sync_copy(bnd.at[pl.ds(pl.multiple_of(w * 16, 8), 16)], bbuf)
        st = bbuf[pl.ds(0, 16)][0]
        pltpu.sync_copy(bnd.at[pl.ds(pl.multiple_of((w + 1) * 16, 8), 16)], bbuf)
        en = bbuf[pl.ds(0, 16)][0]
        kb0 = st // BL
        kb1 = (en + BL - 1) // BL
        nb = kb1 - kb0
        nch = (nb + CHB - 1) // CHB

        def gather(bb, rows, sem):
            return pltpu.async_copy(
                y_hbm.at[src_c.at[pl.ds(bb * BL, BL)]], rows, sem)

        def gwait(rows, sem):
            pltpu.make_async_copy(y_hbm.at[src_c.at[pl.ds(0, BL)]],
                                  rows, sem).wait()

        def accum(rows, bb):
            def grp(g, _):
                d = dst_c[pl.ds(bb * BL + g * 16, 16)]
                m = (d >= lo) & (d < lo + RPW)
                tv = jnp.where(m, d - lo, 0)
                wv = jnp.where(m, w_c[pl.ds(bb * BL + g * 16, 16)], 0.0)
                for t16 in range(16):
                    tt = tv[t16]
                    ws = wv[t16]
                    e = g * 16 + t16
                    if bf16_packed:
                        for q in range(F // 32):
                            wrd = rows[e, pl.ds(q * 16, 16)]
                            ra = jax.lax.bitcast_convert_type(
                                jax.lax.shift_left(wrd, jnp.int32(16)),
                                jnp.float32)
                            rb = jax.lax.bitcast_convert_type(
                                wrd & jnp.int32(-65536), jnp.float32)
                            acc[tt, pl.ds(q * 32, 16)] = (
                                acc[tt, pl.ds(q * 32, 16)] + ra * ws
                            )
                            acc[tt, pl.ds(q * 32 + 16, 16)] = (
                                acc[tt, pl.ds(q * 32 + 16, 16)] + rb * ws
                            )
                    else:
                        for q in range(F // 16):
                            r = rows[e, pl.ds(q * 16, 16)]
                            if relu_gather:
                                r = jnp.maximum(r, 0.0)
                            acc[tt, pl.ds(q * 16, 16)] = (
                                acc[tt, pl.ds(q * 16, 16)] + r * ws
                            )
                return 0
            lax.fori_loop(0, BL // 16, grp, 0)

        def chunk(ci, _):
            cw = pl.multiple_of((kb0 + ci * CHB) * BL, 8)
            pltpu.sync_copy(lsrc.at[pl.ds(cw, CH)], src_c)
            pltpu.sync_copy(ldst.at[pl.ds(cw, CH)], dst_c)
            pltpu.sync_copy(lw.at[pl.ds(cw, CH)], w_c)
            nbc = jnp.minimum(nb - ci * CHB, CHB)

            @pl.when(nbc > 0)
            def _():
                gather(0, rows0, sem0)

            @pl.when(nbc > 1)
            def _():
                gather(1, rows1, sem1)

            def bpair(pb, _):
                b0 = pb * 2

                @pl.when(b0 < nbc)
                def _():
                    gwait(rows0, sem0)
                    accum(rows0, b0)

                    @pl.when(b0 + 2 < nbc)
                    def _():
                        gather(b0 + 2, rows0, sem0)

                @pl.when(b0 + 1 < nbc)
                def _():
                    gwait(rows1, sem1)
                    accum(rows1, b0 + 1)

                    @pl.when(b0 + 3 < nbc)
                    def _():
                        gather(b0 + 3, rows1, sem1)
                return 0
            lax.fori_loop(0, CHB // 2, bpair, 0)
            return 0
        lax.fori_loop(0, nch, chunk, 0)

        pltpu.sync_copy(acc, out_hbm.at[pl.ds(pl.multiple_of(w * RPW, 8), RPW)])

    return sc_layer


def _tc_final(g3, a2, W_rel, W_root, b):
    def body(g_ref, a_ref, wr_ref, wo_ref, b_ref, o_ref):
        h = jnp.maximum(a_ref[...], 0.0)
        o_ref[...] = (
            jnp.dot(g_ref[...], wr_ref[...], preferred_element_type=jnp.float32)
            + jnp.dot(h, wo_ref[...], preferred_element_type=jnp.float32)
            + b_ref[...]
        )

    return pl.pallas_call(
        body,
        grid=(NC * MB,),
        in_specs=[
            pl.BlockSpec((MBLK, H2), lambda m: (m, 0)),
            pl.BlockSpec((MBLK, H2), lambda m: (m, 0)),
            pl.BlockSpec((H2, OUT), lambda m: (0, 0)),
            pl.BlockSpec((H2, OUT), lambda m: (0, 0)),
            pl.BlockSpec((1, OUT), lambda m: (0, 0)),
        ],
        out_specs=pl.BlockSpec((MBLK, OUT), lambda m: (m, 0)),
        out_shape=jax.ShapeDtypeStruct((N, OUT), jnp.float32),
    )(g3, a2, W_rel, W_root, b.reshape(1, OUT))


def kernel(x, edge_index, edge_weight, batch,
           W1_rel, b1, W1_root, W2_rel, b2, W2_root, W3_rel, b3, W3_root):
    src = edge_index[0].astype(jnp.int32)
    dst = edge_index[1].astype(jnp.int32)
    w = edge_weight.astype(jnp.float32)
    order = jnp.argsort(dst)
    lsrc = src[order]
    ldst = dst[order]
    lw = w[order]
    bounds = jnp.searchsorted(
        ldst, (jnp.arange(NW + 1) * RPW).astype(jnp.int32)).astype(jnp.int32)
    bnd = jnp.repeat(bounds, 16)

    g32 = jnp.arange(H) // 32
    j16 = (jnp.arange(H) % 32) % 16
    odd = ((jnp.arange(H) % 32) >= 16).astype(jnp.int32)
    perm = jnp.zeros((H,), jnp.int32).at[32 * g32 + 2 * j16 + odd].set(
        (32 * g32 + j16 + 16 * odd).astype(jnp.int32))
    W1p = W1_rel[:, perm]
    y1, z1 = _tc_layer(x, W1p, W1_root, b1, first=True, y_bf16=True)
    y1i = jax.lax.bitcast_convert_type(
        y1.reshape(N, H // 2, 2), jnp.int32)
    a1 = _make_sc_layer(H, bf16_packed=True)(y1i, z1, lsrc, ldst, lw, bnd)
    y2, z2 = _tc_layer(a1, W2_rel, W2_root, b2, first=False)
    a2 = _make_sc_layer(H2)(y2, z2, lsrc, ldst, lw, bnd)
    g3 = _make_sc_layer(H2, relu_gather=True, zero_init=True)(
        a2, a2, lsrc, ldst, lw, bnd)
    return _tc_final(g3[:N], a2[:N], W3_rel, W3_root, b3)


# speedup vs baseline: 69.4368x; 1.0032x over previous
import functools

import jax
import jax.numpy as jnp
from jax import lax
from jax.experimental import pallas as pl
from jax.experimental.pallas import tpu as pltpu
from jax.experimental.pallas import tpu_sc as plsc

N = 10000
E = 160000
IN = 1024
H = 256
H2 = 128
OUT = 4
OUTP = 16

NC = 2
NS = 16
NW = NC * NS
RPW = 320
NPAD = NW * RPW

B = 128
NBAT = E // B

MBLK = 1000
MB = 5


def _tc_layer(a, W_rel, W_root, b, first, y_bf16=False):
    Fin = W_rel.shape[0]
    F = W_rel.shape[1]

    def body(a_ref, wr_ref, wo_ref, b_ref, y_ref, z_ref):
        h = a_ref[...]
        if not first:
            h = jnp.maximum(h, 0.0)
        y = jnp.dot(h, wr_ref[...], preferred_element_type=jnp.float32)
        y_ref[...] = y.astype(jnp.bfloat16) if y_bf16 else y
        z_ref[...] = (
            jnp.dot(h, wo_ref[...], preferred_element_type=jnp.float32) + b_ref[...]
        )

    return pl.pallas_call(
        body,
        grid=(NC * MB,),
        in_specs=[
            pl.BlockSpec((MBLK, Fin), lambda m: (m, 0)),
            pl.BlockSpec((Fin, F), lambda m: (0, 0)),
            pl.BlockSpec((Fin, F), lambda m: (0, 0)),
            pl.BlockSpec((1, F), lambda m: (0, 0)),
        ],
        out_specs=[
            pl.BlockSpec((MBLK, F), lambda m: (m, 0)),
            pl.BlockSpec((MBLK, F), lambda m: (m, 0)),
        ],
        out_shape=[
            jax.ShapeDtypeStruct((N, F), jnp.bfloat16 if y_bf16 else jnp.float32),
            jax.ShapeDtypeStruct((NPAD, F), jnp.float32),
        ],
    )(a, W_rel, W_root, b.reshape(1, F))


@functools.cache
def _make_sc_layer(F, relu_gather=False, zero_init=False, bf16_packed=False):
    FW = F // 2 if bf16_packed else F
    BL = 64 if FW >= 256 else 128
    CHB = 32
    CH = CHB * BL
    mesh = plsc.VectorSubcoreMesh(core_axis_name="c", subcore_axis_name="s")

    @functools.partial(
        pl.kernel,
        out_type=jax.ShapeDtypeStruct((NPAD, F), jnp.float32),
        mesh=mesh,
        scratch_types=[
            pltpu.VMEM((RPW, F), jnp.float32),
            pltpu.VMEM((CH,), jnp.int32),
            pltpu.VMEM((CH,), jnp.int32),
            pltpu.VMEM((CH,), jnp.float32),
            pltpu.VMEM((BL, FW), jnp.int32 if bf16_packed else jnp.float32),
            pltpu.VMEM((BL, FW), jnp.int32 if bf16_packed else jnp.float32),
            pltpu.VMEM((16,), jnp.int32),
            pltpu.SemaphoreType.DMA,
            pltpu.SemaphoreType.DMA,
        ],
    )
    def sc_layer(y_hbm, z_hbm, lsrc, ldst, lw, bnd, out_hbm,
                 acc, src_c, dst_c, w_c, rows0, rows1, bbuf, sem0, sem1):
        c = lax.axis_index("c")
        s = lax.axis_index("s")
        w = c * NS + s
        lo = w * RPW

        if zero_init:
            zv = jnp.zeros((16,), jnp.float32)

            def zrow(r, _):
                for q in range(F // 16):
                    acc[r, pl.ds(q * 16, 16)] = zv
                return 0
            lax.fori_loop(0, RPW, zrow, 0)
        else:
            pltpu.sync_copy(z_hbm.at[pl.ds(pl.multiple_of(w * RPW, 8), RPW)], acc)

        pltpu.sync_copy(bnd.at[pl.ds(pl.multiple_of(w * 16, 8), 16)], bbuf)
        st = bbuf[pl.ds(0, 16)][0]
        pltpu.sync_copy(bnd.at[pl.ds(pl.multiple_of((w + 1) * 16, 8), 16)], bbuf)
        en = bbuf[pl.ds(0, 16)][0]
        kb0 = st // BL
        kb1 = (en + BL - 1) // BL
        nb = kb1 - kb0
        nch = (nb + CHB - 1) // CHB

        def gather(bb, rows, sem):
            return pltpu.async_copy(
                y_hbm.at[src_c.at[pl.ds(bb * BL, BL)]], rows, sem)

        def gwait(rows, sem):
            pltpu.make_async_copy(y_hbm.at[src_c.at[pl.ds(0, BL)]],
                                  rows, sem).wait()

        def accum(rows, bb):
            def grp(g, _):
                d = dst_c[pl.ds(bb * BL + g * 16, 16)]
                m = (d >= lo) & (d < lo + RPW)
                tv = jnp.where(m, d - lo, 0)
                wv = jnp.where(m, w_c[pl.ds(bb * BL + g * 16, 16)], 0.0)
                for t16 in range(16):
                    tt = tv[t16]
                    ws = wv[t16]
                    e = g * 16 + t16
                    if bf16_packed:
                        for q in range(F // 32):
                            wrd = rows[e, pl.ds(q * 16, 16)]
                            ra = jax.lax.bitcast_convert_type(
                                jax.lax.shift_left(wrd, jnp.int32(16)),
                                jnp.float32)
                            rb = jax.lax.bitcast_convert_type(
                                wrd & jnp.int32(-65536), jnp.float32)
                            acc[tt, pl.ds(q * 32, 16)] = (
                                acc[tt, pl.ds(q * 32, 16)] + ra * ws
                            )
                            acc[tt, pl.ds(q * 32 + 16, 16)] = (
                                acc[tt, pl.ds(q * 32 + 16, 16)] + rb * ws
                            )
                    else:
                        for q in range(F // 16):
                            r = rows[e, pl.ds(q * 16, 16)]
                            if relu_gather:
                                r = jnp.maximum(r, 0.0)
                            acc[tt, pl.ds(q * 16, 16)] = (
                                acc[tt, pl.ds(q * 16, 16)] + r * ws
                            )
                return 0
            lax.fori_loop(0, BL // 16, grp, 0)

        def chunk(ci, _):
            cw = pl.multiple_of((kb0 + ci * CHB) * BL, 8)
            pltpu.sync_copy(lsrc.at[pl.ds(cw, CH)], src_c)
            pltpu.sync_copy(ldst.at[pl.ds(cw, CH)], dst_c)
            pltpu.sync_copy(lw.at[pl.ds(cw, CH)], w_c)
            nbc = jnp.minimum(nb - ci * CHB, CHB)

            @pl.when(nbc > 0)
            def _():
                gather(0, rows0, sem0)

            @pl.when(nbc > 1)
            def _():
                gather(1, rows1, sem1)

            def bpair(pb, _):
                b0 = pb * 2

                @pl.when(b0 < nbc)
                def _():
                    gwait(rows0, sem0)
                    accum(rows0, b0)

                    @pl.when(b0 + 2 < nbc)
                    def _():
                        gather(b0 + 2, rows0, sem0)

                @pl.when(b0 + 1 < nbc)
                def _():
                    gwait(rows1, sem1)
                    accum(rows1, b0 + 1)

                    @pl.when(b0 + 3 < nbc)
                    def _():
                        gather(b0 + 3, rows1, sem1)
                return 0
            lax.fori_loop(0, CHB // 2, bpair, 0)
            return 0
        lax.fori_loop(0, nch, chunk, 0)

        pltpu.sync_copy(acc, out_hbm.at[pl.ds(pl.multiple_of(w * RPW, 8), RPW)])

    return sc_layer


def _tc_final(g3, a2, W_rel, W_root, b):
    def body(g_ref, a_ref, wr_ref, wo_ref, b_ref, o_ref):
        h = jnp.maximum(a_ref[...], 0.0)
        o_ref[...] = (
            jnp.dot(g_ref[...], wr_ref[...], preferred_element_type=jnp.float32)
            + jnp.dot(h, wo_ref[...], preferred_element_type=jnp.float32)
            + b_ref[...]
        )

    return pl.pallas_call(
        body,
        grid=(NC * MB,),
        in_specs=[
            pl.BlockSpec((MBLK, H2), lambda m: (m, 0)),
            pl.BlockSpec((MBLK, H2), lambda m: (m, 0)),
            pl.BlockSpec((H2, OUT), lambda m: (0, 0)),
            pl.BlockSpec((H2, OUT), lambda m: (0, 0)),
            pl.BlockSpec((1, OUT), lambda m: (0, 0)),
        ],
        out_specs=pl.BlockSpec((MBLK, OUT), lambda m: (m, 0)),
        out_shape=jax.ShapeDtypeStruct((N, OUT), jnp.float32),
    )(g3, a2, W_rel, W_root, b.reshape(1, OUT))


def kernel(x, edge_index, edge_weight, batch,
           W1_rel, b1, W1_root, W2_rel, b2, W2_root, W3_rel, b3, W3_root):
    src = edge_index[0].astype(jnp.int32)
    dst = edge_index[1].astype(jnp.int32)
    w = edge_weight.astype(jnp.float32)
    order = jnp.argsort(dst)
    lsrc = src[order]
    ldst = dst[order]
    lw = w[order]
    bounds = jnp.searchsorted(
        ldst, (jnp.arange(NW + 1) * RPW).astype(jnp.int32)).astype(jnp.int32)
    bnd = jnp.repeat(bounds, 16)

    g32 = jnp.arange(H) // 32
    j16 = (jnp.arange(H) % 32) % 16
    odd = ((jnp.arange(H) % 32) >= 16).astype(jnp.int32)
    perm = jnp.zeros((H,), jnp.int32).at[32 * g32 + 2 * j16 + odd].set(
        (32 * g32 + j16 + 16 * odd).astype(jnp.int32))
    W1p = W1_rel[:, perm]
    y1, z1 = _tc_layer(x, W1p, W1_root, b1, first=True, y_bf16=True)
    y1i = jax.lax.bitcast_convert_type(y1.reshape(N, H // 2, 2), jnp.int32)
    a1 = _make_sc_layer(H, bf16_packed=True)(y1i, z1, lsrc, ldst, lw, bnd)
    y2, z2 = _tc_layer(a1, W2_rel, W2_root, b2, first=False)
    a2 = _make_sc_layer(H2)(y2, z2, lsrc, ldst, lw, bnd)
    g3 = _make_sc_layer(H2, relu_gather=True, zero_init=True)(
        a2, a2, lsrc, ldst, lw, bnd)
    return _tc_final(g3, a2, W3_rel, W3_root, b3)
